# baseline (speedup 1.0000x reference)
import functools

import jax
import jax.numpy as jnp
from jax import lax
from jax.experimental import pallas as pl
from jax.experimental.pallas import tpu as pltpu
from jax.experimental.pallas import tpu_sc as plsc

_TC_ROWS = 2560


def _tc_gaines_div(d_ref, o_ref):
    o_ref[...] = (d_ref[0] + d_ref[1] > 0.0).astype(jnp.float32)


def _sc_gaines_div(row0, sc_rows, cols, rows_per_w, stripe, num_cores):
    nch = rows_per_w // stripe
    assert nch % 2 == 0

    mesh = plsc.VectorSubcoreMesh(core_axis_name="c", subcore_axis_name="s")

    @functools.partial(
        pl.kernel,
        mesh=mesh,
        out_type=jax.ShapeDtypeStruct((sc_rows, cols), jnp.float32),
        scratch_types=[
            pltpu.VMEM((2, stripe, cols), jnp.float32),
            pltpu.VMEM((2, stripe, cols), jnp.float32),
            pltpu.VMEM((2, stripe, cols), jnp.float32),
            pltpu.SemaphoreType.DMA((2,)),
            pltpu.SemaphoreType.DMA((2,)),
        ],
    )
    def sc_k(d_hbm, out_hbm, va, vb, vo, sem_in, sem_out):
        wid = lax.axis_index("s") * num_cores + lax.axis_index("c")
        base = wid * rows_per_w

        def start_in(j, slot):
            row = row0 + base + j * stripe
            pltpu.async_copy(d_hbm.at[0, pl.ds(row, stripe), :], va.at[slot],
                             sem_in.at[slot])
            pltpu.async_copy(d_hbm.at[1, pl.ds(row, stripe), :], vb.at[slot],
                             sem_in.at[slot])

        def wait_in(slot):
            pltpu.make_async_copy(d_hbm.at[0, pl.ds(row0, stripe), :],
                                  va.at[slot], sem_in.at[slot]).wait()
            pltpu.make_async_copy(d_hbm.at[0, pl.ds(row0, stripe), :],
                                  vb.at[slot], sem_in.at[slot]).wait()

        def wait_out(slot):
            pltpu.make_async_copy(vo.at[slot],
                                  out_hbm.at[pl.ds(base, stripe), :],
                                  sem_out.at[slot]).wait()

        start_in(0, 0)
        start_in(1, 1)

        def step(g, carry):
            for slot in range(2):
                j = g * 2 + slot
                wait_in(slot)

                @pl.when(g > 0)
                def _():
                    wait_out(slot)

                for r in range(stripe):

                    @plsc.parallel_loop(0, cols, 16, unroll=8)
                    def _(k):
                        s = va[slot, r, pl.ds(k, 16)] + vb[slot, r, pl.ds(k, 16)]
                        vo[slot, r, pl.ds(k, 16)] = jnp.where(s > 0.0, 1.0, 0.0)

                pltpu.async_copy(
                    vo.at[slot],
                    out_hbm.at[pl.ds(base + j * stripe, stripe), :],
                    sem_out.at[slot])

                @pl.when(j + 2 < nch)
                def _():
                    start_in(j + 2, slot)
            return carry

        lax.fori_loop(0, nch // 2, step, 0)
        wait_out(0)
        wait_out(1)

    return sc_k


def kernel(dividend, divisor):
    del divisor
    _, rows, cols = dividend.shape
    info = plsc.get_sparse_core_info()
    nw = info.num_cores * info.num_subcores

    tc_rows = _TC_ROWS
    sc_rows = rows - tc_rows
    rows_per_w = sc_rows // nw
    stripe = 8

    out_sc = _sc_gaines_div(tc_rows, sc_rows, cols, rows_per_w,
                            stripe, info.num_cores)(dividend)

    block_rows = 512
    out_tc = pl.pallas_call(
        _tc_gaines_div,
        grid=(tc_rows // block_rows,),
        in_specs=[pl.BlockSpec((2, block_rows, cols), lambda i: (0, i, 0))],
        out_specs=pl.BlockSpec((block_rows, cols), lambda i: (i, 0)),
        out_shape=jax.ShapeDtypeStruct((rows, cols), jnp.float32),
    )(dividend)

    return jax.lax.dynamic_update_slice(out_tc, out_sc, (tc_rows, 0))


# speedup vs baseline: 1.8938x; 1.8938x over previous
import jax
import jax.numpy as jnp
from jax.experimental import pallas as pl


def _gaines_div_kernel(a_ref, b_ref, o_ref):
    o_ref[...] = (a_ref[...] + b_ref[...] > 0.0).astype(jnp.float32)


def kernel(dividend, divisor):
    del divisor
    _, rows, cols = dividend.shape
    flat = dividend.reshape(2 * rows, cols)
    block_rows = 512
    nblk = rows // block_rows
    off = nblk
    return pl.pallas_call(
        _gaines_div_kernel,
        grid=(nblk,),
        in_specs=[
            pl.BlockSpec((block_rows, cols), lambda i: (i, 0)),
            pl.BlockSpec((block_rows, cols), lambda i, o=off: (i + o, 0)),
        ],
        out_specs=pl.BlockSpec((block_rows, cols), lambda i: (i, 0)),
        out_shape=jax.ShapeDtypeStruct((rows, cols), jnp.float32),
    )(flat, flat)
